# baseline (speedup 1.0000x reference)
import jax
import jax.numpy as jnp
from jax.experimental import pallas as pl
from jax.experimental.pallas import tpu as pltpu

N_E = 1024
D = 64
N_ROWS = 16 * 32 * 32
BLK = 512
N_B = 16
N_J = 1024 // BLK
COMMIT = 0.25


def _vq_block_kernel(in_ref, w2_ref, w2b_ref, wtb_ref, enc_ref, idx_ref,
                     qt_ref, loss_ref, ppl_ref, sse_acc, cnt_acc, bb_ref,
                     iota_ref):
    bi = pl.program_id(0)
    j = pl.program_id(1)
    step = bi * N_J + j

    @pl.when(step == 0)
    def _precompute():
        w2 = w2_ref[...]
        b = jnp.sum(w2 * w2, axis=1, keepdims=True) * 0.25
        bb_ref[...] = jnp.broadcast_to(b, (N_E, BLK))
        iota_ref[...] = jax.lax.broadcasted_iota(jnp.int32, (N_E, BLK), 0)

    xt = in_ref[0, :, pl.ds(j * BLK, BLK)]
    xt_bf = xt.astype(jnp.bfloat16)

    a = jnp.sum(xt * xt, axis=0, keepdims=True)
    scores2t = jax.lax.dot_general(
        w2b_ref[...], xt_bf, (((1,), (0,)), ((), ())),
        preferred_element_type=jnp.float32)
    dist = (a + bb_ref[...]) - scores2t

    iota = iota_ref[...]
    mval = jnp.min(dist, axis=0, keepdims=True)
    idx = jnp.min(jnp.where(dist == mval, iota, N_E), axis=0)
    idx_ref[...] = idx[None, None, :]

    onehot_t = (iota == idx[None, :]).astype(jnp.float32)
    enc_row = onehot_t.T
    enc_ref[...] = enc_row
    onehot_bf = onehot_t.astype(jnp.bfloat16)

    qt = jax.lax.dot_general(
        wtb_ref[...], onehot_bf, (((1,), (0,)), ((), ())),
        preferred_element_type=jnp.float32)
    qt_ref[0, :, pl.ds(j * BLK, BLK)] = xt + (qt - xt)

    diff = qt - xt
    part = jnp.sum(diff * diff, keepdims=True)
    cnt = jax.lax.dot_general(
        onehot_bf, jnp.ones((BLK, 1), jnp.bfloat16),
        (((1,), (0,)), ((), ())),
        preferred_element_type=jnp.float32)

    @pl.when(step == 0)
    def _init():
        sse_acc[...] = part
        cnt_acc[...] = cnt

    @pl.when(step > 0)
    def _accum():
        sse_acc[...] += part
        cnt_acc[...] += cnt

    @pl.when(step == N_B * N_J - 1)
    def _finalize():
        m = sse_acc[...] * (1.0 / float(N_ROWS * D))
        loss_ref[...] = m + COMMIT * m
        avg = cnt_acc[...] * (1.0 / float(N_ROWS))
        ent = jnp.sum(avg * jnp.log(avg + 1e-10), axis=0, keepdims=True)
        ppl_ref[...] = jnp.exp(-ent)


def kernel(inputs, W):
    x = inputs.reshape(16, D, 32 * 32)
    w2 = W + W
    w2b = w2.astype(jnp.bfloat16)
    wtb = W.T.astype(jnp.bfloat16)

    enc, idx, qt, loss, ppl = pl.pallas_call(
        _vq_block_kernel,
        grid=(N_B, N_J),
        in_specs=[
            pl.BlockSpec((1, D, 1024), lambda b, j: (b, 0, 0)),
            pl.BlockSpec((N_E, D), lambda b, j: (0, 0)),
            pl.BlockSpec((N_E, D), lambda b, j: (0, 0)),
            pl.BlockSpec((D, N_E), lambda b, j: (0, 0)),
        ],
        out_specs=[
            pl.BlockSpec((BLK, N_E), lambda b, j: (b * N_J + j, 0)),
            pl.BlockSpec((1, 1, BLK), lambda b, j: (b * N_J + j, 0, 0)),
            pl.BlockSpec((1, D, 1024), lambda b, j: (b, 0, 0)),
            pl.BlockSpec((1, 1), lambda b, j: (0, 0)),
            pl.BlockSpec((1, 1), lambda b, j: (0, 0)),
        ],
        out_shape=[
            jax.ShapeDtypeStruct((N_ROWS, N_E), jnp.float32),
            jax.ShapeDtypeStruct((N_ROWS // BLK, 1, BLK), jnp.int32),
            jax.ShapeDtypeStruct((16, D, 32 * 32), jnp.float32),
            jax.ShapeDtypeStruct((1, 1), jnp.float32),
            jax.ShapeDtypeStruct((1, 1), jnp.float32),
        ],
        scratch_shapes=[
            pltpu.VMEM((1, 1), jnp.float32),
            pltpu.VMEM((N_E, 1), jnp.float32),
            pltpu.VMEM((N_E, BLK), jnp.float32),
            pltpu.VMEM((N_E, BLK), jnp.int32),
        ],
    )(x, w2, w2b, wtb)

    quantized_out = qt.reshape(16, D, 32, 32)
    return (loss[0, 0], quantized_out, ppl[0, 0], enc,
            idx.reshape(N_ROWS, 1))


# speedup vs baseline: 1.1623x; 1.1623x over previous
import jax
import jax.numpy as jnp
from jax.experimental import pallas as pl
from jax.experimental.pallas import tpu as pltpu

N_E = 1024
D = 64
N_ROWS = 16 * 32 * 32
BLK = 1024
N_B = 16
N_J = 1024 // BLK
COMMIT = 0.25


def _vq_block_kernel(in_ref, w2_ref, wt_ref, enc_ref, idx_ref, qt_ref,
                     loss_ref, ppl_ref, sse_acc, cnt_acc, bb_ref, iota_ref):
    bi = pl.program_id(0)
    j = pl.program_id(1)
    step = bi * N_J + j

    @pl.when(step == 0)
    def _precompute():
        w2 = w2_ref[...]
        b = jnp.sum(w2 * w2, axis=1, keepdims=True) * 0.25
        bb_ref[...] = jnp.broadcast_to(b, (N_E, BLK))
        iota_ref[...] = jax.lax.broadcasted_iota(jnp.int32, (N_E, BLK), 0)

    xt = in_ref[0, :, pl.ds(j * BLK, BLK)]
    wt = wt_ref[...]

    a = jnp.sum(xt * xt, axis=0, keepdims=True)
    scores2t = jax.lax.dot_general(
        w2_ref[...], xt, (((1,), (0,)), ((), ())),
        preferred_element_type=jnp.float32)
    dist = (a + bb_ref[...]) - scores2t

    iota = iota_ref[...]
    mval = jnp.min(dist, axis=0, keepdims=True)
    idx = jnp.min(jnp.where(dist == mval, iota, N_E), axis=0)
    idx_ref[...] = idx[None, None, :]

    onehot_t = (iota == idx[None, :]).astype(jnp.float32)
    enc_row = onehot_t.T
    enc_ref[...] = enc_row

    qt = jax.lax.dot_general(
        wt, onehot_t, (((1,), (0,)), ((), ())),
        preferred_element_type=jnp.float32)
    qt_ref[0, :, pl.ds(j * BLK, BLK)] = xt + (qt - xt)

    diff = qt - xt
    part = jnp.sum(diff * diff, keepdims=True)
    cnt = jnp.sum(enc_row, axis=0, keepdims=True)

    @pl.when(step == 0)
    def _init():
        sse_acc[...] = part
        cnt_acc[...] = cnt

    @pl.when(step > 0)
    def _accum():
        sse_acc[...] += part
        cnt_acc[...] += cnt

    @pl.when(step == N_B * N_J - 1)
    def _finalize():
        m = sse_acc[...] * (1.0 / float(N_ROWS * D))
        loss_ref[...] = m + COMMIT * m
        avg = cnt_acc[...] * (1.0 / float(N_ROWS))
        ent = jnp.sum(avg * jnp.log(avg + 1e-10), axis=1, keepdims=True)
        ppl_ref[...] = jnp.exp(-ent)


def kernel(inputs, W):
    x = inputs.reshape(16, D, 32 * 32)
    w2 = W + W
    wt = W.T

    enc, idx, qt, loss, ppl = pl.pallas_call(
        _vq_block_kernel,
        grid=(N_B, N_J),
        in_specs=[
            pl.BlockSpec((1, D, 1024), lambda b, j: (b, 0, 0)),
            pl.BlockSpec((N_E, D), lambda b, j: (0, 0)),
            pl.BlockSpec((D, N_E), lambda b, j: (0, 0)),
        ],
        out_specs=[
            pl.BlockSpec((BLK, N_E), lambda b, j: (b * N_J + j, 0)),
            pl.BlockSpec((1, 1, BLK), lambda b, j: (b * N_J + j, 0, 0)),
            pl.BlockSpec((1, D, 1024), lambda b, j: (b, 0, 0)),
            pl.BlockSpec((1, 1), lambda b, j: (0, 0)),
            pl.BlockSpec((1, 1), lambda b, j: (0, 0)),
        ],
        out_shape=[
            jax.ShapeDtypeStruct((N_ROWS, N_E), jnp.float32),
            jax.ShapeDtypeStruct((N_ROWS // BLK, 1, BLK), jnp.int32),
            jax.ShapeDtypeStruct((16, D, 32 * 32), jnp.float32),
            jax.ShapeDtypeStruct((1, 1), jnp.float32),
            jax.ShapeDtypeStruct((1, 1), jnp.float32),
        ],
        scratch_shapes=[
            pltpu.VMEM((1, 1), jnp.float32),
            pltpu.VMEM((1, N_E), jnp.float32),
            pltpu.VMEM((N_E, BLK), jnp.float32),
            pltpu.VMEM((N_E, BLK), jnp.int32),
        ],
    )(x, w2, wt)

    quantized_out = qt.reshape(16, D, 32, 32)
    return (loss[0, 0], quantized_out, ppl[0, 0], enc,
            idx.reshape(N_ROWS, 1))
